# baseline (speedup 1.0000x reference)
import functools

import jax
import jax.numpy as jnp
from jax import lax
from jax.experimental import pallas as pl
from jax.experimental.pallas import tpu as pltpu
from jax.experimental.pallas import tpu_sc as plsc

_N_CODES = 8192
_K = 256
_HW = 1024
_BN = 512
_N_TILES = _N_CODES // _BN


def _vq_argmin_body(z_ref, emb_ref, idx_ref, emb_n_ref):
    img = pl.program_id(0)

    @pl.when(img == 0)
    def _():
        e = emb_ref[...]
        emb_n_ref[...] = e / jnp.sqrt(
            jnp.sum(e * e, axis=1, keepdims=True) + 1e-12)

    z = z_ref[0]
    zn = z / jnp.sqrt(jnp.sum(z * z, axis=0, keepdims=True) + 1e-12)
    sum_z = jnp.sum(zn * zn, axis=0, keepdims=True)

    def step(n, carry):
        cur_min, cur_idx = carry
        en = emb_n_ref[pl.ds(n * _BN, _BN), :]
        sum_e = jnp.sum(en * en, axis=1, keepdims=True)
        s = lax.dot_general(en, zn, (((1,), (0,)), ((), ())),
                            preferred_element_type=jnp.float32)
        d = (sum_z + sum_e) - 2.0 * s
        tmin = jnp.min(d, axis=0, keepdims=True)
        targ = jnp.argmin(d, axis=0)[None, :].astype(jnp.int32) + n * _BN
        better = tmin < cur_min
        return (jnp.where(better, tmin, cur_min),
                jnp.where(better, targ, cur_idx))

    init = (jnp.full((1, _HW), jnp.inf, jnp.float32),
            jnp.zeros((1, _HW), jnp.int32))
    _, idx = lax.fori_loop(0, _N_TILES, step, init)
    idx_ref[0] = idx


def _tc_argmin(z4, emb):
    return pl.pallas_call(
        _vq_argmin_body,
        grid=(z4.shape[0],),
        in_specs=[
            pl.BlockSpec((1, _K, _HW), lambda i: (i, 0, 0)),
            pl.BlockSpec((_N_CODES, _K), lambda i: (0, 0)),
        ],
        out_specs=pl.BlockSpec((1, 1, _HW), lambda i: (i, 0, 0)),
        out_shape=jax.ShapeDtypeStruct((z4.shape[0], 1, _HW), jnp.int32),
        scratch_shapes=[pltpu.VMEM((_N_CODES, _K), jnp.float32)],
    )(z4, emb)


_NW = 32
_B_PER_W = 256
_CHUNK = 128


def _sc_gather_body(emb_hbm, idx_hbm, out_hbm, idx_v, rows_v, sem):
    wid = lax.axis_index("s") * 2 + lax.axis_index("c")
    base = wid * _B_PER_W
    for j in range(_B_PER_W // _CHUNK):
        pltpu.sync_copy(idx_hbm.at[pl.ds(base + j * _CHUNK, _CHUNK)],
                        idx_v.at[j])
        pltpu.async_copy(emb_hbm.at[idx_v.at[j]],
                         rows_v.at[pl.ds(j * _CHUNK, _CHUNK)], sem).wait()
    pltpu.sync_copy(rows_v, out_hbm.at[pl.ds(base, _B_PER_W)])


def _sc_gather(emb, idx_flat):
    mesh = plsc.VectorSubcoreMesh(core_axis_name="c", subcore_axis_name="s")
    return pl.kernel(
        _sc_gather_body,
        out_type=jax.ShapeDtypeStruct((_NW * _B_PER_W, _K), jnp.float32),
        mesh=mesh,
        scratch_types=[
            pltpu.VMEM((_B_PER_W // _CHUNK, _CHUNK), jnp.int32),
            pltpu.VMEM((_B_PER_W, _K), jnp.float32),
            pltpu.SemaphoreType.DMA,
        ],
    )(emb, idx_flat)


def kernel(z, embedding_weight):
    b, c, h, w = z.shape
    z4 = z.reshape(b, c, h * w)
    idx = _tc_argmin(z4, embedding_weight)
    idx_flat = idx.reshape(-1)
    zq_rows = _sc_gather(embedding_weight, idx_flat)
    z_q = zq_rows.reshape(b, h, w, c).transpose(0, 3, 1, 2)
    return z_q, idx_flat


# speedup vs baseline: 1.0529x; 1.0529x over previous
import functools

import jax
import jax.numpy as jnp
from jax import lax
from jax.experimental import pallas as pl
from jax.experimental.pallas import tpu as pltpu
from jax.experimental.pallas import tpu_sc as plsc

_N_CODES = 8192
_K = 256
_HW = 1024
_BN = 512
_N_TILES = _N_CODES // _BN


def _vq_argmin_body(z_ref, emb_ref, idx_ref, emb_n_ref, sum_e_ref,
                    s0_ref, s1_ref):
    img = pl.program_id(0)

    @pl.when(img == 0)
    def _():
        e = emb_ref[...]
        en = e / jnp.sqrt(jnp.sum(e * e, axis=1, keepdims=True) + 1e-12)
        emb_n_ref[...] = en
        sum_e_ref[...] = jnp.sum(en * en, axis=1, keepdims=True)

    z = z_ref[0]
    zn = z / jnp.sqrt(jnp.sum(z * z, axis=0, keepdims=True) + 1e-12)
    sum_z = jnp.sum(zn * zn, axis=0, keepdims=True)

    def dot_tile(n):
        en = emb_n_ref[pl.ds(n * _BN, _BN), :]
        return lax.dot_general(en, zn, (((1,), (0,)), ((), ())),
                               preferred_element_type=jnp.float32)

    def consume(s_ref, n, carry):
        cur_min, cur_idx = carry
        sum_e = sum_e_ref[pl.ds(n * _BN, _BN), :]
        d = (sum_z + sum_e) - 2.0 * s_ref[...]
        tmin = jnp.min(d, axis=0, keepdims=True)
        targ = jnp.argmin(d, axis=0)[None, :].astype(jnp.int32) + n * _BN
        better = tmin < cur_min
        return (jnp.where(better, tmin, cur_min),
                jnp.where(better, targ, cur_idx))

    s0_ref[...] = dot_tile(0)

    def step(m, carry):
        n = 2 * m
        s1_ref[...] = dot_tile(n + 1)
        carry = consume(s0_ref, n, carry)
        s0_ref[...] = dot_tile((n + 2) % _N_TILES)
        return consume(s1_ref, n + 1, carry)

    init = (jnp.full((1, _HW), jnp.inf, jnp.float32),
            jnp.zeros((1, _HW), jnp.int32))
    _, idx = lax.fori_loop(0, _N_TILES // 2, step, init)
    idx_ref[0] = idx


def _tc_argmin(z4, emb):
    return pl.pallas_call(
        _vq_argmin_body,
        grid=(z4.shape[0],),
        in_specs=[
            pl.BlockSpec((1, _K, _HW), lambda i: (i, 0, 0)),
            pl.BlockSpec((_N_CODES, _K), lambda i: (0, 0)),
        ],
        out_specs=pl.BlockSpec((1, 1, _HW), lambda i: (i, 0, 0)),
        out_shape=jax.ShapeDtypeStruct((z4.shape[0], 1, _HW), jnp.int32),
        scratch_shapes=[pltpu.VMEM((_N_CODES, _K), jnp.float32),
                        pltpu.VMEM((_N_CODES, 1), jnp.float32),
                        pltpu.VMEM((_BN, _HW), jnp.float32),
                        pltpu.VMEM((_BN, _HW), jnp.float32)],
    )(z4, emb)


_NW = 32
_B_PER_W = 256
_CHUNK = 128


def _sc_gather_body(emb_hbm, idx_hbm, out_hbm, idx_v, rows_v, sem):
    wid = lax.axis_index("s") * 2 + lax.axis_index("c")
    base = wid * _B_PER_W
    for j in range(_B_PER_W // _CHUNK):
        pltpu.sync_copy(idx_hbm.at[pl.ds(base + j * _CHUNK, _CHUNK)],
                        idx_v.at[j])
        pltpu.async_copy(emb_hbm.at[idx_v.at[j]],
                         rows_v.at[pl.ds(j * _CHUNK, _CHUNK)], sem).wait()
    pltpu.sync_copy(rows_v, out_hbm.at[pl.ds(base, _B_PER_W)])


def _sc_gather(emb, idx_flat):
    mesh = plsc.VectorSubcoreMesh(core_axis_name="c", subcore_axis_name="s")
    return pl.kernel(
        _sc_gather_body,
        out_type=jax.ShapeDtypeStruct((_NW * _B_PER_W, _K), jnp.float32),
        mesh=mesh,
        scratch_types=[
            pltpu.VMEM((_B_PER_W // _CHUNK, _CHUNK), jnp.int32),
            pltpu.VMEM((_B_PER_W, _K), jnp.float32),
            pltpu.SemaphoreType.DMA,
        ],
    )(emb, idx_flat)


def kernel(z, embedding_weight):
    b, c, h, w = z.shape
    z4 = z.reshape(b, c, h * w)
    idx = _tc_argmin(z4, embedding_weight)
    idx_flat = idx.reshape(-1)
    zq_rows = _sc_gather(embedding_weight, idx_flat)
    z_q = zq_rows.reshape(b, h, w, c).transpose(0, 3, 1, 2)
    return z_q, idx_flat
